# baseline (speedup 1.0000x reference)
import numpy as np

import jax
import jax.numpy as jnp
from jax import lax
from jax.experimental import pallas as pl
from jax.experimental.pallas import tpu as pltpu
from jax.experimental.pallas import tpu_sc as plsc

B = 4
N = 16384
F = 128
MAX_GT = 8192
TOP_K = 3

L = 16
KV = F // L
CH = 128
GR = 4
NW = 32
CHUNKS_PER_BATCH = MAX_GT // CH
TOTAL_CHUNKS = B * CHUNKS_PER_BATCH
STEPS = TOTAL_CHUNKS // NW

_NEG = np.float32(-3.4028234663852886e38)
_IMIN = np.int32(-2147483648)

_GATHER_DNUMS = lax.GatherDimensionNumbers(
    offset_dims=(), collapsed_slice_dims=(0,), start_index_map=(0,)
)


def _rot(x, idx):
    return lax.gather(
        x,
        idx,
        dimension_numbers=_GATHER_DNUMS,
        slice_sizes=(1,),
        mode=lax.GatherScatterMode.PROMISE_IN_BOUNDS,
    )


def _sc_body(
    soft_hbm, params_hbm, out_hbm,
    pvec, vin0, vin1, vout0, vout1, negbuf, sin0, sin1, sout0, sout1,
):
    wid = lax.axis_index("s") * 2 + lax.axis_index("c")
    pltpu.sync_copy(params_hbm, pvec)
    lanes = lax.iota(jnp.int32, L)
    pv = pvec[...]
    rotidx = [((lanes + s) & (L - 1)).reshape(L, 1) for s in (8, 4, 2, 1)]
    vins = [vin0, vin1]
    vouts = [vout0, vout1]
    sin = [sin0, sin1]
    sout = [sout0, sout1]

    def extract(idx):
        return jnp.max(jnp.where(lanes == idx, pv, _IMIN))

    negv = jnp.full((L,), _NEG)
    none = jnp.full((L,), jnp.float32(-1.0))
    zero = jnp.zeros((L,), jnp.float32)

    def fill_neg(r, _):
        for k in range(KV):
            negbuf[r, pl.ds(L * k, L)] = none
        return 0

    def row_compute(vin, vout, rin, rout, validv):
        x = [vin[rin, pl.ds(L * k, L)] for k in range(KV)]
        a1 = x[0]
        a2 = negv
        a3 = negv
        for k in range(1, KV):
            t1 = jnp.maximum(a1, x[k])
            c2 = jnp.minimum(a1, x[k])
            t2 = jnp.maximum(a2, c2)
            c3 = jnp.minimum(a2, c2)
            a3 = jnp.maximum(a3, c3)
            a1 = t1
            a2 = t2
        for i, s in enumerate((8, 4, 2)):
            b1 = _rot(a1, rotidx[i])
            b2 = _rot(a2, rotidx[i])
            b3 = _rot(a3, rotidx[i])
            l1 = jnp.maximum(a1, b3)
            l2 = jnp.maximum(a2, b2)
            l3 = jnp.maximum(a3, b1)
            u = jnp.maximum(l1, l2)
            v = jnp.minimum(l1, l2)
            w = jnp.maximum(v, l3)
            xm = jnp.minimum(v, l3)
            a1 = jnp.maximum(u, w)
            a2 = jnp.minimum(u, w)
            a3 = xm
        b1 = _rot(a1, rotidx[3])
        b2 = _rot(a2, rotidx[3])
        b3 = _rot(a3, rotidx[3])
        l1 = jnp.maximum(a1, b3)
        l2 = jnp.maximum(a2, b2)
        l3 = jnp.maximum(a3, b1)
        thr = jnp.minimum(jnp.minimum(l1, l2), l3)
        for k in range(KV):
            vout[rout, pl.ds(L * k, L)] = jnp.where(
                validv, jnp.where(x[k] >= thr, x[k], zero), none
            )

    def make_group_body(vin, vout, dshift, vcb):
        def group_body(g, _):
            r0 = g * GR
            for r in range(GR):
                rout = r0 + r
                validv = jnp.full((L,), rout, jnp.int32) < vcb
                rin = jnp.minimum(rout + dshift, CH)
                row_compute(vin, vout, rin, rout, validv)
            return 0

        return group_body

    scal = []
    for t in range(STEPS):
        if t % 2 == 0:
            cg = jnp.int32(t * NW) + wid
        else:
            cg = jnp.int32(t * NW + NW - 1) - wid
        bi = cg // CHUNKS_PER_BATCH
        cl = cg % CHUNKS_PER_BATCH
        start = extract(bi)
        cnt = extract(bi + B)
        src = start + cl * CH
        vc = jnp.clip(jnp.minimum(cnt, MAX_GT) - cl * CH, 0, CH)
        srcc = jnp.minimum(src, N - CH)
        dshift = src - srcc
        scal.append((cg, vc, srcc, dshift))

    def start_in(t):
        return pltpu.async_copy(
            soft_hbm.at[pl.ds(scal[t][2], CH)],
            vins[t % 2].at[pl.ds(0, CH)],
            sin[t % 2],
        )

    hin = [start_in(0)]
    lax.fori_loop(0, CH, fill_neg, 0)

    for t in range(STEPS):
        cg, vc, srcc, dshift = scal[t]
        if t + 1 < STEPS:
            hin.append(start_in(t + 1))
        hin[t].wait()
        if t >= 2:
            pltpu.make_async_copy(
                negbuf,
                out_hbm.at[pl.ds(scal[t - 2][0] * CH, CH)],
                sout[t % 2],
            ).wait()
        vin_t = vins[t % 2]
        vout_t = vouts[t % 2]
        out_at = out_hbm.at[pl.ds(cg * CH, CH)]

        @pl.when(vc > 0)
        def _(vin_t=vin_t, vout_t=vout_t, vc=vc, dshift=dshift, out_at=out_at, t=t):
            vcb = jnp.full((L,), vc, jnp.int32)
            lax.fori_loop(
                0, CH // GR, make_group_body(vin_t, vout_t, dshift, vcb), 0
            )
            pltpu.async_copy(vout_t, out_at, sout[t % 2])

        @pl.when(vc <= 0)
        def _(out_at=out_at, t=t):
            pltpu.async_copy(negbuf, out_at, sout[t % 2])

    for t in (STEPS - 2, STEPS - 1):
        pltpu.make_async_copy(
            negbuf,
            out_hbm.at[pl.ds(scal[t][0] * CH, CH)],
            sout[t % 2],
        ).wait()


@jax.jit
def kernel(soft_weight, gt_boxes_batch_ids, gt_boxes_count):
    del gt_boxes_batch_ids
    counts = gt_boxes_count[:, 0].astype(jnp.int32)
    starts = jnp.concatenate(
        [jnp.zeros((1,), jnp.int32), jnp.cumsum(counts)[:-1].astype(jnp.int32)]
    )
    params = jnp.concatenate(
        [starts, counts, jnp.zeros((L - 2 * B,), jnp.int32)]
    )

    mesh = plsc.VectorSubcoreMesh(core_axis_name="c", subcore_axis_name="s")
    out = pl.kernel(
        _sc_body,
        out_type=jax.ShapeDtypeStruct((B * MAX_GT, F), jnp.float32),
        mesh=mesh,
        compiler_params=pltpu.CompilerParams(
            use_tc_tiling_on_sc=False, needs_layout_passes=False
        ),
        scratch_types=[
            pltpu.VMEM((L,), jnp.int32),
            pltpu.VMEM((CH + GR, F), jnp.float32),
            pltpu.VMEM((CH + GR, F), jnp.float32),
            pltpu.VMEM((CH, F), jnp.float32),
            pltpu.VMEM((CH, F), jnp.float32),
            pltpu.VMEM((CH, F), jnp.float32),
            pltpu.SemaphoreType.DMA,
            pltpu.SemaphoreType.DMA,
            pltpu.SemaphoreType.DMA,
            pltpu.SemaphoreType.DMA,
        ],
    )(soft_weight, params)
    return out.reshape(B, MAX_GT, F)


# speedup vs baseline: 1.0512x; 1.0512x over previous
import numpy as np

import jax
import jax.numpy as jnp
from jax import lax
from jax.experimental import pallas as pl
from jax.experimental.pallas import tpu as pltpu
from jax.experimental.pallas import tpu_sc as plsc

B = 4
N = 16384
F = 128
MAX_GT = 8192
TOP_K = 3

L = 16
KV = F // L
CH = 128
GR = 4
NW = 32
CHUNKS_PER_BATCH = MAX_GT // CH
TOTAL_CHUNKS = B * CHUNKS_PER_BATCH
STEPS = TOTAL_CHUNKS // NW

_NEG = np.float32(-3.4028234663852886e38)
_IMIN = np.int32(-2147483648)

_GATHER_DNUMS = lax.GatherDimensionNumbers(
    offset_dims=(), collapsed_slice_dims=(0,), start_index_map=(0,)
)


def _rot(x, idx):
    return lax.gather(
        x,
        idx,
        dimension_numbers=_GATHER_DNUMS,
        slice_sizes=(1,),
        mode=lax.GatherScatterMode.PROMISE_IN_BOUNDS,
    )


def _sc_body(
    soft_hbm, params_hbm, out_hbm,
    pvec, vin0, vin1, vout0, vout1, negbuf, sin0, sin1, sout0, sout1,
):
    wid = lax.axis_index("s") * 2 + lax.axis_index("c")
    pltpu.sync_copy(params_hbm, pvec)
    lanes = lax.iota(jnp.int32, L)
    pv = pvec[...]
    rotidx = [((lanes + s) & (L - 1)).reshape(L, 1) for s in (8, 4, 2, 1)]
    vins = [vin0, vin1]
    vouts = [vout0, vout1]
    sin = [sin0, sin1]
    sout = [sout0, sout1]

    def extract(idx):
        return jnp.max(jnp.where(lanes == idx, pv, _IMIN))

    negv = jnp.full((L,), _NEG)
    none = jnp.full((L,), jnp.float32(-1.0))
    zero = jnp.zeros((L,), jnp.float32)

    def fill_neg(r, _):
        for k in range(KV):
            negbuf[r, pl.ds(L * k, L)] = none
        return 0

    def row_compute(vin, vout, rin, rout):
        x = [vin[rin, pl.ds(L * k, L)] for k in range(KV)]
        a1 = x[0]
        a2 = negv
        a3 = negv
        for k in range(1, KV):
            t1 = jnp.maximum(a1, x[k])
            c2 = jnp.minimum(a1, x[k])
            t2 = jnp.maximum(a2, c2)
            c3 = jnp.minimum(a2, c2)
            a3 = jnp.maximum(a3, c3)
            a1 = t1
            a2 = t2
        for i, s in enumerate((8, 4, 2)):
            b1 = _rot(a1, rotidx[i])
            b2 = _rot(a2, rotidx[i])
            b3 = _rot(a3, rotidx[i])
            l1 = jnp.maximum(a1, b3)
            l2 = jnp.maximum(a2, b2)
            l3 = jnp.maximum(a3, b1)
            u = jnp.maximum(l1, l2)
            v = jnp.minimum(l1, l2)
            w = jnp.maximum(v, l3)
            xm = jnp.minimum(v, l3)
            a1 = jnp.maximum(u, w)
            a2 = jnp.minimum(u, w)
            a3 = xm
        b1 = _rot(a1, rotidx[3])
        b2 = _rot(a2, rotidx[3])
        b3 = _rot(a3, rotidx[3])
        l1 = jnp.maximum(a1, b3)
        l2 = jnp.maximum(a2, b2)
        l3 = jnp.maximum(a3, b1)
        thr = jnp.minimum(jnp.minimum(l1, l2), l3)
        for k in range(KV):
            vout[rout, pl.ds(L * k, L)] = jnp.where(x[k] >= thr, x[k], zero)

    def make_fill_row(vout):
        def fill_row(r, _):
            for k in range(KV):
                vout[r, pl.ds(L * k, L)] = none
            return 0

        return fill_row

    scal = []
    for t in range(STEPS):
        if t % 2 == 0:
            cg = jnp.int32(t * NW) + wid
        else:
            cg = jnp.int32(t * NW + NW - 1) - wid
        bi = cg // CHUNKS_PER_BATCH
        cl = cg % CHUNKS_PER_BATCH
        start = extract(bi)
        cnt = extract(bi + B)
        src = start + cl * CH
        vc = jnp.clip(jnp.minimum(cnt, MAX_GT) - cl * CH, 0, CH)
        srcc = jnp.minimum(src, N - CH)
        dshift = src - srcc
        scal.append((cg, vc, srcc, dshift))

    def start_in(t):
        return pltpu.async_copy(
            soft_hbm.at[pl.ds(scal[t][2], CH)],
            vins[t % 2].at[pl.ds(0, CH)],
            sin[t % 2],
        )

    hin = [start_in(0)]
    lax.fori_loop(0, CH, fill_neg, 0)

    for t in range(STEPS):
        cg, vc, srcc, dshift = scal[t]
        if t + 1 < STEPS:
            hin.append(start_in(t + 1))
        hin[t].wait()
        if t >= 2:
            pltpu.make_async_copy(
                negbuf,
                out_hbm.at[pl.ds(scal[t - 2][0] * CH, CH)],
                sout[t % 2],
            ).wait()
        vin_t = vins[t % 2]
        vout_t = vouts[t % 2]
        out_at = out_hbm.at[pl.ds(cg * CH, CH)]

        @pl.when(vc > 0)
        def _(vin_t=vin_t, vout_t=vout_t, vc=vc, dshift=dshift, out_at=out_at, t=t):
            @plsc.parallel_loop(0, vc, step=1, unroll=GR)
            def _row(r):
                row_compute(vin_t, vout_t, r + dshift, r)

            lax.fori_loop(vc, CH, make_fill_row(vout_t), 0)
            pltpu.async_copy(vout_t, out_at, sout[t % 2])

        @pl.when(vc <= 0)
        def _(out_at=out_at, t=t):
            pltpu.async_copy(negbuf, out_at, sout[t % 2])

    for t in (STEPS - 2, STEPS - 1):
        pltpu.make_async_copy(
            negbuf,
            out_hbm.at[pl.ds(scal[t][0] * CH, CH)],
            sout[t % 2],
        ).wait()


@jax.jit
def kernel(soft_weight, gt_boxes_batch_ids, gt_boxes_count):
    del gt_boxes_batch_ids
    counts = gt_boxes_count[:, 0].astype(jnp.int32)
    starts = jnp.concatenate(
        [jnp.zeros((1,), jnp.int32), jnp.cumsum(counts)[:-1].astype(jnp.int32)]
    )
    params = jnp.concatenate(
        [starts, counts, jnp.zeros((L - 2 * B,), jnp.int32)]
    )

    mesh = plsc.VectorSubcoreMesh(core_axis_name="c", subcore_axis_name="s")
    out = pl.kernel(
        _sc_body,
        out_type=jax.ShapeDtypeStruct((B * MAX_GT, F), jnp.float32),
        mesh=mesh,
        compiler_params=pltpu.CompilerParams(
            use_tc_tiling_on_sc=False, needs_layout_passes=False
        ),
        scratch_types=[
            pltpu.VMEM((L,), jnp.int32),
            pltpu.VMEM((CH + GR, F), jnp.float32),
            pltpu.VMEM((CH + GR, F), jnp.float32),
            pltpu.VMEM((CH, F), jnp.float32),
            pltpu.VMEM((CH, F), jnp.float32),
            pltpu.VMEM((CH, F), jnp.float32),
            pltpu.SemaphoreType.DMA,
            pltpu.SemaphoreType.DMA,
            pltpu.SemaphoreType.DMA,
            pltpu.SemaphoreType.DMA,
        ],
    )(soft_weight, params)
    return out.reshape(B, MAX_GT, F)


# speedup vs baseline: 1.0671x; 1.0152x over previous
import numpy as np

import jax
import jax.numpy as jnp
from jax import lax
from jax.experimental import pallas as pl
from jax.experimental.pallas import tpu as pltpu
from jax.experimental.pallas import tpu_sc as plsc

B = 4
N = 16384
F = 128
MAX_GT = 8192
TOP_K = 3

L = 16
KV = F // L
CH = 128
GR = 4
NW = 32
CHUNKS_PER_BATCH = MAX_GT // CH
TOTAL_CHUNKS = B * CHUNKS_PER_BATCH
STEPS = TOTAL_CHUNKS // NW

_NEG = np.float32(-3.4028234663852886e38)
_IMIN = np.int32(-2147483648)

_GATHER_DNUMS = lax.GatherDimensionNumbers(
    offset_dims=(), collapsed_slice_dims=(0,), start_index_map=(0,)
)


def _rot(x, idx):
    return lax.gather(
        x,
        idx,
        dimension_numbers=_GATHER_DNUMS,
        slice_sizes=(1,),
        mode=lax.GatherScatterMode.PROMISE_IN_BOUNDS,
    )


def _sc_body(
    soft_hbm, params_hbm, out_hbm,
    pvec, vin0, vin1, vout0, vout1, negbuf, sin0, sin1, sout0, sout1,
):
    wid = lax.axis_index("s") * 2 + lax.axis_index("c")
    pltpu.sync_copy(params_hbm, pvec)
    lanes = lax.iota(jnp.int32, L)
    pv = pvec[...]
    rotidx = [((lanes + s) & (L - 1)).reshape(L, 1) for s in (8, 4, 2, 1)]
    vins = [vin0, vin1]
    vouts = [vout0, vout1]
    sin = [sin0, sin1]
    sout = [sout0, sout1]

    def extract(idx):
        return jnp.max(jnp.where(lanes == idx, pv, _IMIN))

    negv = jnp.full((L,), _NEG)
    none = jnp.full((L,), jnp.float32(-1.0))
    zero = jnp.zeros((L,), jnp.float32)

    def fill_neg(r, _):
        for k in range(KV):
            negbuf[r, pl.ds(L * k, L)] = none
        return 0

    def top3of4(x1, x2, x3, x4):
        hi1 = jnp.maximum(x1, x2)
        lo1 = jnp.minimum(x1, x2)
        hi2 = jnp.maximum(x3, x4)
        lo2 = jnp.minimum(x3, x4)
        s = jnp.minimum(hi1, hi2)
        t = jnp.maximum(lo1, lo2)
        return (
            jnp.maximum(hi1, hi2),
            jnp.maximum(s, t),
            jnp.minimum(s, t),
        )

    def merge33(a1, a2, a3, b1, b2, b3):
        l1 = jnp.maximum(a1, b3)
        l2 = jnp.maximum(a2, b2)
        l3 = jnp.maximum(a3, b1)
        u = jnp.maximum(l1, l2)
        v = jnp.minimum(l1, l2)
        w = jnp.maximum(v, l3)
        xm = jnp.minimum(v, l3)
        return jnp.maximum(u, w), jnp.minimum(u, w), xm

    def row_compute(vin, vout, rin, rout):
        x = [vin[rin, pl.ds(L * k, L)] for k in range(KV)]
        h1, h2, h3 = top3of4(x[0], x[1], x[2], x[3])
        g1, g2, g3 = top3of4(x[4], x[5], x[6], x[7])
        a1, a2, a3 = merge33(h1, h2, h3, g1, g2, g3)
        for i, s in enumerate((8, 4, 2)):
            b1 = _rot(a1, rotidx[i])
            b2 = _rot(a2, rotidx[i])
            b3 = _rot(a3, rotidx[i])
            a1, a2, a3 = merge33(a1, a2, a3, b1, b2, b3)
        b1 = _rot(a1, rotidx[3])
        b2 = _rot(a2, rotidx[3])
        b3 = _rot(a3, rotidx[3])
        l1 = jnp.maximum(a1, b3)
        l2 = jnp.maximum(a2, b2)
        l3 = jnp.maximum(a3, b1)
        thr = jnp.minimum(jnp.minimum(l1, l2), l3)
        for k in range(KV):
            vout[rout, pl.ds(L * k, L)] = jnp.where(x[k] >= thr, x[k], zero)

    def make_group_body(vin, vout, dshift):
        def group_body(g, _):
            r0 = g * GR
            for r in range(GR):
                row_compute(vin, vout, r0 + r + dshift, r0 + r)
            return 0

        return group_body

    def make_fill_row(vout):
        def fill_row(r, _):
            for k in range(KV):
                vout[r, pl.ds(L * k, L)] = none
            return 0

        return fill_row

    scal = []
    for t in range(STEPS):
        if t % 2 == 0:
            cg = jnp.int32(t * NW) + wid
        else:
            cg = jnp.int32(t * NW + NW - 1) - wid
        bi = cg // CHUNKS_PER_BATCH
        cl = cg % CHUNKS_PER_BATCH
        start = extract(bi)
        cnt = extract(bi + B)
        src = start + cl * CH
        vc = jnp.clip(jnp.minimum(cnt, MAX_GT) - cl * CH, 0, CH)
        srcc = jnp.minimum(src, N - CH)
        dshift = src - srcc
        scal.append((cg, vc, srcc, dshift))

    def start_in(t):
        return pltpu.async_copy(
            soft_hbm.at[pl.ds(scal[t][2], CH)],
            vins[t % 2].at[pl.ds(0, CH)],
            sin[t % 2],
        )

    hin = [start_in(0)]
    lax.fori_loop(0, CH, fill_neg, 0)

    for t in range(STEPS):
        cg, vc, srcc, dshift = scal[t]
        if t + 1 < STEPS:
            hin.append(start_in(t + 1))
        hin[t].wait()
        if t >= 2:
            pltpu.make_async_copy(
                negbuf,
                out_hbm.at[pl.ds(scal[t - 2][0] * CH, CH)],
                sout[t % 2],
            ).wait()
        vin_t = vins[t % 2]
        vout_t = vouts[t % 2]
        out_at = out_hbm.at[pl.ds(cg * CH, CH)]

        @pl.when(vc > 0)
        def _(vin_t=vin_t, vout_t=vout_t, vc=vc, dshift=dshift, out_at=out_at, t=t):
            ngrp = (vc + (GR - 1)) // GR
            lax.fori_loop(0, ngrp, make_group_body(vin_t, vout_t, dshift), 0)
            lax.fori_loop(vc, CH, make_fill_row(vout_t), 0)
            pltpu.async_copy(vout_t, out_at, sout[t % 2])

        @pl.when(vc <= 0)
        def _(out_at=out_at, t=t):
            pltpu.async_copy(negbuf, out_at, sout[t % 2])

    for t in (STEPS - 2, STEPS - 1):
        pltpu.make_async_copy(
            negbuf,
            out_hbm.at[pl.ds(scal[t][0] * CH, CH)],
            sout[t % 2],
        ).wait()


@jax.jit
def kernel(soft_weight, gt_boxes_batch_ids, gt_boxes_count):
    del gt_boxes_batch_ids
    counts = gt_boxes_count[:, 0].astype(jnp.int32)
    starts = jnp.concatenate(
        [jnp.zeros((1,), jnp.int32), jnp.cumsum(counts)[:-1].astype(jnp.int32)]
    )
    params = jnp.concatenate(
        [starts, counts, jnp.zeros((L - 2 * B,), jnp.int32)]
    )

    mesh = plsc.VectorSubcoreMesh(core_axis_name="c", subcore_axis_name="s")
    out = pl.kernel(
        _sc_body,
        out_type=jax.ShapeDtypeStruct((B * MAX_GT, F), jnp.float32),
        mesh=mesh,
        compiler_params=pltpu.CompilerParams(
            use_tc_tiling_on_sc=False, needs_layout_passes=False
        ),
        scratch_types=[
            pltpu.VMEM((L,), jnp.int32),
            pltpu.VMEM((CH + GR, F), jnp.float32),
            pltpu.VMEM((CH + GR, F), jnp.float32),
            pltpu.VMEM((CH, F), jnp.float32),
            pltpu.VMEM((CH, F), jnp.float32),
            pltpu.VMEM((CH, F), jnp.float32),
            pltpu.SemaphoreType.DMA,
            pltpu.SemaphoreType.DMA,
            pltpu.SemaphoreType.DMA,
            pltpu.SemaphoreType.DMA,
        ],
    )(soft_weight, params)
    return out.reshape(B, MAX_GT, F)


# speedup vs baseline: 1.0767x; 1.0090x over previous
import numpy as np

import jax
import jax.numpy as jnp
from jax import lax
from jax.experimental import pallas as pl
from jax.experimental.pallas import tpu as pltpu
from jax.experimental.pallas import tpu_sc as plsc

B = 4
N = 16384
F = 128
MAX_GT = 8192
TOP_K = 3

L = 16
KV = F // L
CH = 128
GR = 4
NW = 32
CHUNKS_PER_BATCH = MAX_GT // CH
TOTAL_CHUNKS = B * CHUNKS_PER_BATCH
STEPS = TOTAL_CHUNKS // NW

_NEG = np.float32(-3.4028234663852886e38)
_IMIN = np.int32(-2147483648)

_GATHER_DNUMS = lax.GatherDimensionNumbers(
    offset_dims=(), collapsed_slice_dims=(0,), start_index_map=(0,)
)


def _rot(x, idx):
    return lax.gather(
        x,
        idx,
        dimension_numbers=_GATHER_DNUMS,
        slice_sizes=(1,),
        mode=lax.GatherScatterMode.PROMISE_IN_BOUNDS,
    )


def _sc_body(
    soft_hbm, params_hbm, out_hbm,
    pvec, vin0, vin1, vout0, vout1, negbuf, sin0, sin1, sout0, sout1,
):
    wid = lax.axis_index("s") * 2 + lax.axis_index("c")
    pltpu.sync_copy(params_hbm, pvec)
    lanes = lax.iota(jnp.int32, L)
    pv = pvec[...]
    rotidx = [((lanes + s) & (L - 1)).reshape(L, 1) for s in (8, 4, 2, 1)]
    vins = [vin0, vin1]
    vouts = [vout0, vout1]
    sin = [sin0, sin1]
    sout = [sout0, sout1]

    def extract(idx):
        return jnp.max(jnp.where(lanes == idx, pv, _IMIN))

    negv = jnp.full((L,), _NEG)
    none = jnp.full((L,), jnp.float32(-1.0))
    zero = jnp.zeros((L,), jnp.float32)

    def fill_neg(r, _):
        for k in range(KV):
            negbuf[r, pl.ds(L * k, L)] = none
        return 0

    def row_compute(vin, vout, rin, rout):
        x = [vin[rin, pl.ds(L * k, L)] for k in range(KV)]
        a1 = x[0]
        a2 = negv
        a3 = negv
        for k in range(1, KV):
            t1 = jnp.maximum(a1, x[k])
            c2 = jnp.minimum(a1, x[k])
            t2 = jnp.maximum(a2, c2)
            c3 = jnp.minimum(a2, c2)
            a3 = jnp.maximum(a3, c3)
            a1 = t1
            a2 = t2
        for i, s in enumerate((8, 4, 2)):
            b1 = _rot(a1, rotidx[i])
            b2 = _rot(a2, rotidx[i])
            b3 = _rot(a3, rotidx[i])
            l1 = jnp.maximum(a1, b3)
            l2 = jnp.maximum(a2, b2)
            l3 = jnp.maximum(a3, b1)
            u = jnp.maximum(l1, l2)
            v = jnp.minimum(l1, l2)
            w = jnp.maximum(v, l3)
            xm = jnp.minimum(v, l3)
            a1 = jnp.maximum(u, w)
            a2 = jnp.minimum(u, w)
            a3 = xm
        b1 = _rot(a1, rotidx[3])
        b2 = _rot(a2, rotidx[3])
        b3 = _rot(a3, rotidx[3])
        l1 = jnp.maximum(a1, b3)
        l2 = jnp.maximum(a2, b2)
        l3 = jnp.maximum(a3, b1)
        thr = jnp.minimum(jnp.minimum(l1, l2), l3)
        for k in range(KV):
            vout[rout, pl.ds(L * k, L)] = jnp.where(x[k] >= thr, x[k], zero)

    def make_group_body(vin, vout, dshift):
        def group_body(g, _):
            r0 = g * GR
            for r in range(GR):
                row_compute(vin, vout, r0 + r + dshift, r0 + r)
            return 0

        return group_body

    def make_fill_row(vout):
        def fill_row(r, _):
            for k in range(KV):
                vout[r, pl.ds(L * k, L)] = none
            return 0

        return fill_row

    scal = []
    for t in range(STEPS):
        if t % 2 == 0:
            cg = jnp.int32(t * NW) + wid
        else:
            cg = jnp.int32(t * NW + NW - 1) - wid
        bi = cg // CHUNKS_PER_BATCH
        cl = cg % CHUNKS_PER_BATCH
        start = extract(bi)
        cnt = extract(bi + B)
        src = start + cl * CH
        vc = jnp.clip(jnp.minimum(cnt, MAX_GT) - cl * CH, 0, CH)
        srcc = jnp.minimum(src, N - CH)
        dshift = src - srcc
        scal.append((cg, vc, srcc, dshift))

    def start_in(t):
        return pltpu.async_copy(
            soft_hbm.at[pl.ds(scal[t][2], CH)],
            vins[t % 2].at[pl.ds(0, CH)],
            sin[t % 2],
        )

    hin = [start_in(0)]
    lax.fori_loop(0, CH, fill_neg, 0)

    for t in range(STEPS):
        cg, vc, srcc, dshift = scal[t]
        if t + 1 < STEPS:
            hin.append(start_in(t + 1))
        hin[t].wait()
        if t >= 2:
            pltpu.make_async_copy(
                negbuf,
                out_hbm.at[pl.ds(scal[t - 2][0] * CH, CH)],
                sout[t % 2],
            ).wait()
        vin_t = vins[t % 2]
        vout_t = vouts[t % 2]
        out_at = out_hbm.at[pl.ds(cg * CH, CH)]

        @pl.when(vc > 0)
        def _(vin_t=vin_t, vout_t=vout_t, vc=vc, dshift=dshift, out_at=out_at, t=t):
            ngrp = (vc + (GR - 1)) // GR
            lax.fori_loop(0, ngrp, make_group_body(vin_t, vout_t, dshift), 0)
            lax.fori_loop(vc, CH, make_fill_row(vout_t), 0)
            pltpu.async_copy(vout_t, out_at, sout[t % 2])

        @pl.when(vc <= 0)
        def _(out_at=out_at, t=t):
            pltpu.async_copy(negbuf, out_at, sout[t % 2])

    for t in (STEPS - 2, STEPS - 1):
        pltpu.make_async_copy(
            negbuf,
            out_hbm.at[pl.ds(scal[t][0] * CH, CH)],
            sout[t % 2],
        ).wait()


@jax.jit
def kernel(soft_weight, gt_boxes_batch_ids, gt_boxes_count):
    del gt_boxes_batch_ids
    counts = gt_boxes_count[:, 0].astype(jnp.int32)
    starts = jnp.concatenate(
        [jnp.zeros((1,), jnp.int32), jnp.cumsum(counts)[:-1].astype(jnp.int32)]
    )
    params = jnp.concatenate(
        [starts, counts, jnp.zeros((L - 2 * B,), jnp.int32)]
    )

    mesh = plsc.VectorSubcoreMesh(core_axis_name="c", subcore_axis_name="s")
    out = pl.kernel(
        _sc_body,
        out_type=jax.ShapeDtypeStruct((B * MAX_GT, F), jnp.float32),
        mesh=mesh,
        compiler_params=pltpu.CompilerParams(
            use_tc_tiling_on_sc=False, needs_layout_passes=False
        ),
        scratch_types=[
            pltpu.VMEM((L,), jnp.int32),
            pltpu.VMEM((CH + GR, F), jnp.float32),
            pltpu.VMEM((CH + GR, F), jnp.float32),
            pltpu.VMEM((CH, F), jnp.float32),
            pltpu.VMEM((CH, F), jnp.float32),
            pltpu.VMEM((CH, F), jnp.float32),
            pltpu.SemaphoreType.DMA,
            pltpu.SemaphoreType.DMA,
            pltpu.SemaphoreType.DMA,
            pltpu.SemaphoreType.DMA,
        ],
    )(soft_weight, params)
    return out.reshape(B, MAX_GT, F)
